# baseline (speedup 1.0000x reference)
import functools

import jax
import jax.numpy as jnp
from jax import lax
from jax.experimental import pallas as pl
from jax.experimental.pallas import tpu as pltpu
from jax.experimental.pallas import tpu_sc as plsc

N = 10000
D = 128
E = 320000
NC, NS = 2, 16
NW = NC * NS
B = 128
NB = 2 * (-(-E // (NW * B * 2)))
EPAD = NW * NB * B
NPAD = 10240
RPT = NPAD // NS
NBTOT = EPAD // B
HC = 32
NB0 = 160
NB1 = NBTOT // NS - NB0

_mesh = plsc.VectorSubcoreMesh(core_axis_name="c", subcore_axis_name="s")


@functools.partial(
    pl.kernel,
    out_type=jax.ShapeDtypeStruct((NC, NPAD, D), jnp.float32),
    mesh=_mesh,
    scratch_types=[
        pltpu.VMEM((HC, B), jnp.int32),
        pltpu.VMEM((HC, B), jnp.int32),
        pltpu.VMEM((B, D), jnp.float32),
        pltpu.VMEM((B, D), jnp.float32),
        pltpu.VMEM_SHARED((NPAD, D), jnp.float32),
        pltpu.SemaphoreType.DMA,
        pltpu.SemaphoreType.DMA,
    ],
)
def _seg_sum(h_hbm, src_hbm, dst_hbm, zero_hbm, out_hbm,
             src_v, dst_v, rows0_v, rows1_v, acc_sh, sem0, sem1):
    c = lax.axis_index("c")
    s = lax.axis_index("s")
    row0 = s * RPT
    pltpu.sync_copy(zero_hbm.at[pl.ds(row0, RPT)], acc_sh.at[pl.ds(row0, RPT)])
    plsc.subcore_barrier()

    start = jnp.where(c == 0, s * NB0, NS * NB0 + s * NB1)
    n_chunks = jnp.where(c == 0, NB0 // HC, NB1 // HC)

    def chunk(ci, _):
        b0 = start + ci * HC
        pltpu.sync_copy(src_hbm.at[pl.ds(b0, HC)], src_v)
        pltpu.sync_copy(dst_hbm.at[pl.ds(b0, HC)], dst_v)
        pltpu.async_copy(h_hbm.at[src_v.at[0]], rows0_v, sem0)
        pltpu.async_copy(h_hbm.at[src_v.at[1]], rows1_v, sem1)

        def body(k, _):
            i = 2 * k
            pltpu.make_async_copy(h_hbm.at[src_v.at[i]], rows0_v, sem0).wait()
            pltpu.sync_copy(rows0_v, acc_sh.at[dst_v.at[i]], add=True)
            pltpu.async_copy(h_hbm.at[src_v.at[i + 2]], rows0_v, sem0)
            pltpu.make_async_copy(h_hbm.at[src_v.at[i + 1]], rows1_v, sem1).wait()
            pltpu.sync_copy(rows1_v, acc_sh.at[dst_v.at[i + 1]], add=True)
            pltpu.async_copy(h_hbm.at[src_v.at[i + 3]], rows1_v, sem1)
            return ()

        lax.fori_loop(0, HC // 2 - 1, body, ())
        pltpu.make_async_copy(h_hbm.at[src_v.at[HC - 2]], rows0_v, sem0).wait()
        pltpu.sync_copy(rows0_v, acc_sh.at[dst_v.at[HC - 2]], add=True)
        pltpu.make_async_copy(h_hbm.at[src_v.at[HC - 1]], rows1_v, sem1).wait()
        pltpu.sync_copy(rows1_v, acc_sh.at[dst_v.at[HC - 1]], add=True)
        return ()

    lax.fori_loop(0, n_chunks, chunk, ())
    plsc.subcore_barrier()
    pltpu.sync_copy(acc_sh.at[pl.ds(row0, RPT)], out_hbm.at[c, pl.ds(row0, RPT)])


@functools.partial(
    pl.kernel,
    out_type=jax.ShapeDtypeStruct((NC, NPAD, D), jnp.float32),
    mesh=_mesh,
    scratch_types=[
        pltpu.VMEM((NB, B), jnp.int32),
        pltpu.VMEM((B, D), jnp.float32),
        pltpu.VMEM_SHARED((NPAD, D), jnp.float32),
    ],
)
def _degree(dst_hbm, ones_hbm, zero_hbm, out_hbm, dst_v, ones_v, acc_sh):
    c = lax.axis_index("c")
    s = lax.axis_index("s")
    wid = c * NS + s
    row0 = s * RPT
    pltpu.sync_copy(zero_hbm.at[pl.ds(row0, RPT)], acc_sh.at[pl.ds(row0, RPT)])
    pltpu.sync_copy(dst_hbm.at[pl.ds(wid * NB, NB)], dst_v)
    pltpu.sync_copy(ones_hbm, ones_v)
    plsc.subcore_barrier()

    def body(i, _):
        pltpu.sync_copy(ones_v, acc_sh.at[dst_v.at[i]], add=True)
        return ()

    lax.fori_loop(0, NB, body, ())
    plsc.subcore_barrier()
    pltpu.sync_copy(acc_sh.at[pl.ds(row0, RPT)], out_hbm.at[c, pl.ds(row0, RPT)])


RB = 2000


def _dinv(d0_ref, d1_ref):
    return lax.rsqrt(d0_ref[:, 0:1] + d1_ref[:, 0:1] + 1.0)


def _h1_body(x_ref, w_ref, d0_ref, d1_ref, o_ref):
    h = jnp.dot(x_ref[...], w_ref[...], preferred_element_type=jnp.float32)
    o_ref[...] = h * _dinv(d0_ref, d1_ref)


def _mid_body(s0_ref, s1_ref, hp_ref, d0_ref, d1_ref, b_ref, w_ref,
              e1_ref, h2_ref):
    dinv = _dinv(d0_ref, d1_ref)
    e1 = dinv * (s0_ref[...] + s1_ref[...] + hp_ref[...]) + b_ref[...]
    e1_ref[...] = e1
    h2_ref[...] = jnp.dot(e1, w_ref[...],
                          preferred_element_type=jnp.float32) * dinv


def _fin_body(s0_ref, s1_ref, hp_ref, d0_ref, d1_ref, b_ref, x_ref, e1_ref,
              e2_ref, tot_ref):
    dinv = _dinv(d0_ref, d1_ref)
    e2 = dinv * (s0_ref[...] + s1_ref[...] + hp_ref[...]) + b_ref[...]
    e2_ref[...] = e2
    tot_ref[...] = x_ref[...] + e1_ref[...] + e2


_row = pl.BlockSpec((RB, D), lambda i: (i, 0))
_deg = pl.BlockSpec((RB, 16), lambda i: (i, 0))
_mat = pl.BlockSpec((D, D), lambda i: (0, 0))
_bias = pl.BlockSpec((1, D), lambda i: (0, 0))
_fout = jax.ShapeDtypeStruct((N, D), jnp.float32)
_grid = (N // RB,)


def kernel(item_emb, W0, b0, W1, b1, edge_index):
    x0 = item_emb[:N]
    pad = EPAD - E
    src3 = jnp.concatenate(
        [edge_index[0], jnp.zeros((pad,), jnp.int32)]).reshape(NBTOT, B)
    dst3 = jnp.concatenate(
        [edge_index[1], jnp.full((pad,), N + 16, jnp.int32)]).reshape(NBTOT, B)
    zeros_big = jnp.zeros((NPAD, D), jnp.float32)
    ones_b = jnp.ones((B, D), jnp.float32)

    degp = _degree(dst3, ones_b, zeros_big)
    degp0, degp1 = degp[0, :N, :16], degp[1, :N, :16]

    h1p = pl.pallas_call(
        _h1_body,
        grid=_grid,
        in_specs=[_row, _mat, _deg, _deg],
        out_specs=_row,
        out_shape=_fout,
    )(x0, W0, degp0, degp1)

    s1p = _seg_sum(h1p, src3, dst3, zeros_big)
    e1, h2p = pl.pallas_call(
        _mid_body,
        grid=_grid,
        in_specs=[_row, _row, _row, _deg, _deg, _bias, _mat],
        out_specs=(_row, _row),
        out_shape=(_fout, _fout),
    )(s1p[0, :N], s1p[1, :N], h1p, degp0, degp1, b0.reshape(1, D), W1)

    s2p = _seg_sum(h2p, src3, dst3, zeros_big)
    e2, total = pl.pallas_call(
        _fin_body,
        grid=_grid,
        in_specs=[_row, _row, _row, _deg, _deg, _bias, _row, _row],
        out_specs=(_row, _row),
        out_shape=(_fout, _fout),
    )(s2p[0, :N], s2p[1, :N], h2p, degp0, degp1, b1.reshape(1, D), x0, e1)

    return (total, x0, e1, e2)


# speedup vs baseline: 1.0664x; 1.0664x over previous
import functools

import jax
import jax.numpy as jnp
from jax import lax
from jax.experimental import pallas as pl
from jax.experimental.pallas import tpu as pltpu
from jax.experimental.pallas import tpu_sc as plsc

N = 10000
D = 128
E = 320000
NC, NS = 2, 16
NW = NC * NS
B = 128
NB = 2 * (-(-E // (NW * B * 2)))
EPAD = NW * NB * B
NPAD = 10240
RPT = NPAD // NS
NBTOT = EPAD // B
HC = 32
NB0 = 0
NB1 = NBTOT // NS - NB0

_mesh = plsc.VectorSubcoreMesh(core_axis_name="c", subcore_axis_name="s")


@functools.partial(
    pl.kernel,
    out_type=jax.ShapeDtypeStruct((NC, NPAD, D), jnp.float32),
    mesh=_mesh,
    scratch_types=[
        pltpu.VMEM((HC, B), jnp.int32),
        pltpu.VMEM((HC, B), jnp.int32),
        pltpu.VMEM((B, D), jnp.float32),
        pltpu.VMEM((B, D), jnp.float32),
        pltpu.VMEM_SHARED((NPAD, D), jnp.float32),
        pltpu.SemaphoreType.DMA,
        pltpu.SemaphoreType.DMA,
    ],
)
def _seg_sum(h_hbm, src_hbm, dst_hbm, zero_hbm, out_hbm,
             src_v, dst_v, rows0_v, rows1_v, acc_sh, sem0, sem1):
    c = lax.axis_index("c")
    s = lax.axis_index("s")
    row0 = s * RPT
    pltpu.sync_copy(zero_hbm.at[pl.ds(row0, RPT)], acc_sh.at[pl.ds(row0, RPT)])
    plsc.subcore_barrier()

    start = jnp.where(c == 0, s * NB0, NS * NB0 + s * NB1)
    n_chunks = jnp.where(c == 0, NB0 // HC, NB1 // HC)

    def chunk(ci, _):
        b0 = start + ci * HC
        pltpu.sync_copy(src_hbm.at[pl.ds(b0, HC)], src_v)
        pltpu.sync_copy(dst_hbm.at[pl.ds(b0, HC)], dst_v)
        pltpu.async_copy(h_hbm.at[src_v.at[0]], rows0_v, sem0)
        pltpu.async_copy(h_hbm.at[src_v.at[1]], rows1_v, sem1)

        def body(k, _):
            i = 2 * k
            pltpu.make_async_copy(h_hbm.at[src_v.at[i]], rows0_v, sem0).wait()
            pltpu.sync_copy(rows0_v, acc_sh.at[dst_v.at[i]], add=True)
            pltpu.async_copy(h_hbm.at[src_v.at[i + 2]], rows0_v, sem0)
            pltpu.make_async_copy(h_hbm.at[src_v.at[i + 1]], rows1_v, sem1).wait()
            pltpu.sync_copy(rows1_v, acc_sh.at[dst_v.at[i + 1]], add=True)
            pltpu.async_copy(h_hbm.at[src_v.at[i + 3]], rows1_v, sem1)
            return ()

        lax.fori_loop(0, HC // 2 - 1, body, ())
        pltpu.make_async_copy(h_hbm.at[src_v.at[HC - 2]], rows0_v, sem0).wait()
        pltpu.sync_copy(rows0_v, acc_sh.at[dst_v.at[HC - 2]], add=True)
        pltpu.make_async_copy(h_hbm.at[src_v.at[HC - 1]], rows1_v, sem1).wait()
        pltpu.sync_copy(rows1_v, acc_sh.at[dst_v.at[HC - 1]], add=True)
        return ()

    lax.fori_loop(0, n_chunks, chunk, ())
    plsc.subcore_barrier()
    pltpu.sync_copy(acc_sh.at[pl.ds(row0, RPT)], out_hbm.at[c, pl.ds(row0, RPT)])


@functools.partial(
    pl.kernel,
    out_type=jax.ShapeDtypeStruct((NC, NPAD, D), jnp.float32),
    mesh=_mesh,
    scratch_types=[
        pltpu.VMEM((NB, B), jnp.int32),
        pltpu.VMEM((B, D), jnp.float32),
        pltpu.VMEM_SHARED((NPAD, D), jnp.float32),
    ],
)
def _degree(dst_hbm, ones_hbm, zero_hbm, out_hbm, dst_v, ones_v, acc_sh):
    c = lax.axis_index("c")
    s = lax.axis_index("s")
    wid = c * NS + s
    row0 = s * RPT
    pltpu.sync_copy(zero_hbm.at[pl.ds(row0, RPT)], acc_sh.at[pl.ds(row0, RPT)])
    pltpu.sync_copy(dst_hbm.at[pl.ds(wid * NB, NB)], dst_v)
    pltpu.sync_copy(ones_hbm, ones_v)
    plsc.subcore_barrier()

    def body(i, _):
        pltpu.sync_copy(ones_v, acc_sh.at[dst_v.at[i]], add=True)
        return ()

    lax.fori_loop(0, NB, body, ())
    plsc.subcore_barrier()
    pltpu.sync_copy(acc_sh.at[pl.ds(row0, RPT)], out_hbm.at[c, pl.ds(row0, RPT)])


RB = 2000


def _dinv(d0_ref, d1_ref):
    return lax.rsqrt(d0_ref[:, 0:1] + d1_ref[:, 0:1] + 1.0)


def _h1_body(x_ref, w_ref, d0_ref, d1_ref, o_ref):
    h = jnp.dot(x_ref[...], w_ref[...], preferred_element_type=jnp.float32)
    o_ref[...] = h * _dinv(d0_ref, d1_ref)


def _mid_body(s0_ref, s1_ref, hp_ref, d0_ref, d1_ref, b_ref, w_ref,
              e1_ref, h2_ref):
    dinv = _dinv(d0_ref, d1_ref)
    e1 = dinv * (s0_ref[...] + s1_ref[...] + hp_ref[...]) + b_ref[...]
    e1_ref[...] = e1
    h2_ref[...] = jnp.dot(e1, w_ref[...],
                          preferred_element_type=jnp.float32) * dinv


def _fin_body(s0_ref, s1_ref, hp_ref, d0_ref, d1_ref, b_ref, x_ref, e1_ref,
              e2_ref, tot_ref):
    dinv = _dinv(d0_ref, d1_ref)
    e2 = dinv * (s0_ref[...] + s1_ref[...] + hp_ref[...]) + b_ref[...]
    e2_ref[...] = e2
    tot_ref[...] = x_ref[...] + e1_ref[...] + e2


_row = pl.BlockSpec((RB, D), lambda i: (i, 0))
_deg = pl.BlockSpec((RB, 16), lambda i: (i, 0))
_mat = pl.BlockSpec((D, D), lambda i: (0, 0))
_bias = pl.BlockSpec((1, D), lambda i: (0, 0))
_fout = jax.ShapeDtypeStruct((N, D), jnp.float32)
_grid = (N // RB,)


def kernel(item_emb, W0, b0, W1, b1, edge_index):
    x0 = item_emb[:N]
    pad = EPAD - E
    src3 = jnp.concatenate(
        [edge_index[0], jnp.zeros((pad,), jnp.int32)]).reshape(NBTOT, B)
    dst3 = jnp.concatenate(
        [edge_index[1], jnp.full((pad,), N + 16, jnp.int32)]).reshape(NBTOT, B)
    zeros_big = jnp.zeros((NPAD, D), jnp.float32)
    ones_b = jnp.ones((B, D), jnp.float32)

    degp = _degree(dst3, ones_b, zeros_big)
    degp0, degp1 = degp[0, :N, :16], degp[1, :N, :16]

    h1p = pl.pallas_call(
        _h1_body,
        grid=_grid,
        in_specs=[_row, _mat, _deg, _deg],
        out_specs=_row,
        out_shape=_fout,
    )(x0, W0, degp0, degp1)

    s1p = _seg_sum(h1p, src3, dst3, zeros_big)
    e1, h2p = pl.pallas_call(
        _mid_body,
        grid=_grid,
        in_specs=[_row, _row, _row, _deg, _deg, _bias, _mat],
        out_specs=(_row, _row),
        out_shape=(_fout, _fout),
    )(s1p[0, :N], s1p[1, :N], h1p, degp0, degp1, b0.reshape(1, D), W1)

    s2p = _seg_sum(h2p, src3, dst3, zeros_big)
    e2, total = pl.pallas_call(
        _fin_body,
        grid=_grid,
        in_specs=[_row, _row, _row, _deg, _deg, _bias, _row, _row],
        out_specs=(_row, _row),
        out_shape=(_fout, _fout),
    )(s2p[0, :N], s2p[1, :N], h2p, degp0, degp1, b1.reshape(1, D), x0, e1)

    return (total, x0, e1, e2)


# speedup vs baseline: 1.3539x; 1.2696x over previous
import functools

import jax
import jax.numpy as jnp
from jax import lax
from jax.experimental import pallas as pl
from jax.experimental.pallas import tpu as pltpu
from jax.experimental.pallas import tpu_sc as plsc

N = 10000
D = 128
E = 320000
NC, NS = 2, 16
NW = NC * NS
B = 128
NB = 2 * (-(-E // (NW * B * 2)))
EPAD = NW * NB * B
NPAD = 10240
RPT = NPAD // NS
NBTOT = EPAD // B
HC = 32
NB0 = 80
NB1 = NBTOT // NS - NB0

_mesh = plsc.VectorSubcoreMesh(core_axis_name="c", subcore_axis_name="s")


@functools.partial(
    pl.kernel,
    out_type=jax.ShapeDtypeStruct((NC, NPAD, D), jnp.float32),
    mesh=_mesh,
    scratch_types=[
        pltpu.VMEM((HC, B), jnp.int32),
        pltpu.VMEM((HC, B), jnp.int32),
        pltpu.VMEM((B, D), jnp.float32),
        pltpu.VMEM((B, D), jnp.float32),
        pltpu.VMEM_SHARED((NPAD, D), jnp.float32),
        pltpu.SemaphoreType.DMA,
        pltpu.SemaphoreType.DMA,
    ],
)
def _seg_sum(h_hbm, src_hbm, dst_hbm, zero_hbm, out_hbm,
             src_v, dst_v, rows0_v, rows1_v, acc_sh, sem0, sem1):
    c = lax.axis_index("c")
    s = lax.axis_index("s")
    row0 = s * RPT
    pltpu.sync_copy(zero_hbm.at[pl.ds(row0, RPT)], acc_sh.at[pl.ds(row0, RPT)])
    plsc.subcore_barrier()

    start = jnp.where(c == 0, s * NB0, NS * NB0 + s * NB1)
    n_chunks = jnp.where(c == 0, NB0 // HC, NB1 // HC)

    def chunk(ci, _):
        b0 = start + ci * HC
        pltpu.sync_copy(src_hbm.at[pl.ds(b0, HC)], src_v)
        pltpu.sync_copy(dst_hbm.at[pl.ds(b0, HC)], dst_v)

        def body(i, _):
            pltpu.async_copy(h_hbm.at[src_v.at[i]], rows0_v, sem0).wait()
            pltpu.sync_copy(rows0_v, acc_sh.at[dst_v.at[i]], add=True)
            return ()

        lax.fori_loop(0, HC, body, ())
        return ()

    lax.fori_loop(0, n_chunks, chunk, ())
    plsc.subcore_barrier()
    pltpu.sync_copy(acc_sh.at[pl.ds(row0, RPT)], out_hbm.at[c, pl.ds(row0, RPT)])


@functools.partial(
    pl.kernel,
    out_type=jax.ShapeDtypeStruct((NC, NPAD, D), jnp.float32),
    mesh=_mesh,
    scratch_types=[
        pltpu.VMEM((NB, B), jnp.int32),
        pltpu.VMEM((B, D), jnp.float32),
        pltpu.VMEM_SHARED((NPAD, D), jnp.float32),
    ],
)
def _degree(dst_hbm, ones_hbm, zero_hbm, out_hbm, dst_v, ones_v, acc_sh):
    c = lax.axis_index("c")
    s = lax.axis_index("s")
    wid = c * NS + s
    row0 = s * RPT
    pltpu.sync_copy(zero_hbm.at[pl.ds(row0, RPT)], acc_sh.at[pl.ds(row0, RPT)])
    pltpu.sync_copy(dst_hbm.at[pl.ds(wid * NB, NB)], dst_v)
    pltpu.sync_copy(ones_hbm, ones_v)
    plsc.subcore_barrier()

    def body(i, _):
        pltpu.sync_copy(ones_v, acc_sh.at[dst_v.at[i]], add=True)
        return ()

    lax.fori_loop(0, NB, body, ())
    plsc.subcore_barrier()
    pltpu.sync_copy(acc_sh.at[pl.ds(row0, RPT)], out_hbm.at[c, pl.ds(row0, RPT)])


RB = 2000


def _dinv(d0_ref, d1_ref):
    return lax.rsqrt(d0_ref[:, 0:1] + d1_ref[:, 0:1] + 1.0)


def _h1_body(x_ref, w_ref, d0_ref, d1_ref, o_ref):
    h = jnp.dot(x_ref[...], w_ref[...], preferred_element_type=jnp.float32)
    o_ref[...] = h * _dinv(d0_ref, d1_ref)


def _mid_body(s0_ref, s1_ref, hp_ref, d0_ref, d1_ref, b_ref, w_ref,
              e1_ref, h2_ref):
    dinv = _dinv(d0_ref, d1_ref)
    e1 = dinv * (s0_ref[...] + s1_ref[...] + hp_ref[...]) + b_ref[...]
    e1_ref[...] = e1
    h2_ref[...] = jnp.dot(e1, w_ref[...],
                          preferred_element_type=jnp.float32) * dinv


def _fin_body(s0_ref, s1_ref, hp_ref, d0_ref, d1_ref, b_ref, x_ref, e1_ref,
              e2_ref, tot_ref):
    dinv = _dinv(d0_ref, d1_ref)
    e2 = dinv * (s0_ref[...] + s1_ref[...] + hp_ref[...]) + b_ref[...]
    e2_ref[...] = e2
    tot_ref[...] = x_ref[...] + e1_ref[...] + e2


_row = pl.BlockSpec((RB, D), lambda i: (i, 0))
_deg = pl.BlockSpec((RB, 16), lambda i: (i, 0))
_mat = pl.BlockSpec((D, D), lambda i: (0, 0))
_bias = pl.BlockSpec((1, D), lambda i: (0, 0))
_fout = jax.ShapeDtypeStruct((N, D), jnp.float32)
_grid = (N // RB,)


def kernel(item_emb, W0, b0, W1, b1, edge_index):
    x0 = item_emb[:N]
    pad = EPAD - E
    src3 = jnp.concatenate(
        [edge_index[0], jnp.zeros((pad,), jnp.int32)]).reshape(NBTOT, B)
    dst3 = jnp.concatenate(
        [edge_index[1], jnp.full((pad,), N + 16, jnp.int32)]).reshape(NBTOT, B)
    zeros_big = jnp.zeros((NPAD, D), jnp.float32)
    ones_b = jnp.ones((B, D), jnp.float32)

    degp = _degree(dst3, ones_b, zeros_big)
    degp0, degp1 = degp[0, :N, :16], degp[1, :N, :16]

    h1p = pl.pallas_call(
        _h1_body,
        grid=_grid,
        in_specs=[_row, _mat, _deg, _deg],
        out_specs=_row,
        out_shape=_fout,
    )(x0, W0, degp0, degp1)

    s1p = _seg_sum(h1p, src3, dst3, zeros_big)
    e1, h2p = pl.pallas_call(
        _mid_body,
        grid=_grid,
        in_specs=[_row, _row, _row, _deg, _deg, _bias, _mat],
        out_specs=(_row, _row),
        out_shape=(_fout, _fout),
    )(s1p[0, :N], s1p[1, :N], h1p, degp0, degp1, b0.reshape(1, D), W1)

    s2p = _seg_sum(h2p, src3, dst3, zeros_big)
    e2, total = pl.pallas_call(
        _fin_body,
        grid=_grid,
        in_specs=[_row, _row, _row, _deg, _deg, _bias, _row, _row],
        out_specs=(_row, _row),
        out_shape=(_fout, _fout),
    )(s2p[0, :N], s2p[1, :N], h2p, degp0, degp1, b1.reshape(1, D), x0, e1)

    return (total, x0, e1, e2)
